# baseline (speedup 1.0000x reference)
import functools

import jax
import jax.numpy as jnp
from jax import lax
from jax.experimental import pallas as pl
from jax.experimental.pallas import tpu as pltpu
from jax.experimental.pallas import tpu_sc as plsc

D = 128
L = 16
NW = 32
V = 10000
VP = V // 2
NW_ROW = 128
HALF = 64
E = 320000
E_PAD = 327680
EPW = E_PAD // NW
CHUNK = 128
NCHUNK = EPW // CHUNK
STRIPE = 312

_mesh = plsc.VectorSubcoreMesh(core_axis_name="c", subcore_axis_name="s")


@functools.partial(
    pl.kernel,
    out_type=jax.ShapeDtypeStruct((E_PAD,), jnp.float32),
    mesh=_mesh,
    scratch_types=[
        pltpu.VMEM_SHARED((VP, NW_ROW), jnp.int32),
        pltpu.VMEM((CHUNK,), jnp.int32),
        pltpu.VMEM((CHUNK,), jnp.int32),
        pltpu.VMEM((CHUNK,), jnp.int32),
        pltpu.VMEM((CHUNK,), jnp.int32),
        pltpu.VMEM((CHUNK,), jnp.int32),
        pltpu.VMEM((CHUNK,), jnp.int32),
        pltpu.VMEM((CHUNK,), jnp.int32),
        pltpu.VMEM((CHUNK,), jnp.int32),
        pltpu.VMEM((CHUNK,), jnp.int32),
        pltpu.VMEM((CHUNK,), jnp.int32),
        pltpu.VMEM((CHUNK,), jnp.int32),
        pltpu.VMEM((CHUNK,), jnp.int32),
        pltpu.VMEM((CHUNK, NW_ROW), jnp.int32),
        pltpu.VMEM((CHUNK, NW_ROW), jnp.int32),
        pltpu.VMEM((CHUNK, NW_ROW), jnp.int32),
        pltpu.VMEM((CHUNK, NW_ROW), jnp.int32),
        pltpu.VMEM((CHUNK,), jnp.float32),
        pltpu.VMEM((CHUNK,), jnp.float32),
        pltpu.SemaphoreType.DMA,
        pltpu.SemaphoreType.DMA,
        pltpu.SemaphoreType.DMA,
        pltpu.SemaphoreType.DMA,
        pltpu.SemaphoreType.DMA,
        pltpu.SemaphoreType.DMA,
    ],
    compiler_params=pltpu.CompilerParams(needs_layout_passes=False),
)
def _dot_scores(h_hbm, sidx_hbm, didx_hbm, out_hbm,
                tab_sp, is0, id0, is1, id1,
                gs0, gd0, gs1, gd1, ps0, pd0, ps1, pd1,
                s0, d0, s1, d1, o0, o1,
                sem0, sem1, semi0, semi1, semo0, semo1):
    sid = lax.axis_index("s")
    wid = sid * 2 + lax.axis_index("c")
    base = wid * EPW

    @pl.when(sid < 15)
    def _():
        pltpu.sync_copy(h_hbm.at[pl.ds(sid * STRIPE, STRIPE)],
                        tab_sp.at[pl.ds(sid * STRIPE, STRIPE)])

    @pl.when(sid == 15)
    def _():
        pltpu.sync_copy(h_hbm.at[pl.ds(15 * STRIPE, VP - 15 * STRIPE)],
                        tab_sp.at[pl.ds(15 * STRIPE, VP - 15 * STRIPE)])

    plsc.subcore_barrier()

    def idx_stage(c, isb, idb, sem, *, sync=False):
        sl = pl.ds(base + c * CHUNK, CHUNK)
        if sync:
            pltpu.sync_copy(sidx_hbm.at[sl], isb)
            pltpu.sync_copy(didx_hbm.at[sl], idb)
        else:
            pltpu.async_copy(sidx_hbm.at[sl], isb, sem)
            pltpu.async_copy(didx_hbm.at[sl], idb, sem)

    def idx_wait(c, isb, idb, sem):
        sl = pl.ds(base + c * CHUNK, CHUNK)
        pltpu.make_async_copy(sidx_hbm.at[sl], isb, sem).wait()
        pltpu.make_async_copy(didx_hbm.at[sl], idb, sem).wait()

    def split_idx(isb, idb, gsb, gdb, psb, pdb):
        for j in range(CHUNK // L):
            sl = pl.ds(j * L, L)
            v = isb[sl]
            gsb[sl] = lax.shift_right_logical(v, 1)
            psb[sl] = lax.shift_left(jnp.bitwise_and(v, 1), 6)
            w = idb[sl]
            gdb[sl] = lax.shift_right_logical(w, 1)
            pdb[sl] = lax.shift_left(jnp.bitwise_and(w, 1), 6)

    def issue(gsb, gdb, sbuf, dbuf, sem):
        pltpu.async_copy(tab_sp.at[gsb], sbuf, sem)
        pltpu.async_copy(tab_sp.at[gdb], dbuf, sem)

    def wait(gsb, gdb, sbuf, dbuf, sem):
        pltpu.make_async_copy(tab_sp.at[gsb], sbuf, sem).wait()
        pltpu.make_async_copy(tab_sp.at[gdb], dbuf, sem).wait()

    def out_store(c, ob, sem):
        pltpu.async_copy(ob, out_hbm.at[pl.ds(base + c * CHUNK, CHUNK)], sem)

    def out_wait(c, ob, sem):
        pltpu.make_async_copy(ob, out_hbm.at[pl.ds(base + c * CHUNK, CHUNK)],
                              sem).wait()

    def compute(sbuf, dbuf, psb, pdb, ob):
        @plsc.parallel_loop(0, CHUNK, step=L)
        def edge_body(e0):
            lane = lax.iota(jnp.int32, L)
            i15 = jnp.full((L,), L - 1, jnp.int32)
            soffs = psb[pl.ds(e0, L)]
            doffs = pdb[pl.ds(e0, L)]
            res = jnp.zeros((L,), jnp.float32)
            for i in range(L):
                e = e0 + i
                soff = soffs[i]
                doff = doffs[i]
                acc = jnp.zeros((L,), jnp.float32)
                for k in range(HALF // L):
                    sv = plsc.bitcast(sbuf[e, pl.ds(soff + L * k, L)],
                                      jnp.bfloat16)
                    dv = plsc.bitcast(dbuf[e, pl.ds(doff + L * k, L)],
                                      jnp.bfloat16)
                    a, b = plsc.unpack(sv * dv,
                                       format=plsc.PackFormat.INTERLEAVED)
                    acc = acc + a + b
                scn = plsc.cumsum(acc)
                res = jnp.where(lane == i, scn[i15], res)
            ob[pl.ds(e0, L)] = res

    idx_stage(0, is0, id0, semi0, sync=True)
    split_idx(is0, id0, gs0, gd0, ps0, pd0)
    issue(gs0, gd0, s0, d0, sem0)
    idx_stage(1, is1, id1, semi1)

    def pair(cc, carry):
        c = 2 * cc
        idx_wait(c + 1, is1, id1, semi1)
        split_idx(is1, id1, gs1, gd1, ps1, pd1)
        issue(gs1, gd1, s1, d1, sem1)
        wait(gs0, gd0, s0, d0, sem0)

        @pl.when(c + 2 < NCHUNK)
        def _():
            idx_stage(c + 2, is0, id0, semi0)

        @pl.when(cc >= 1)
        def _():
            out_wait(c - 2, o0, semo0)

        compute(s0, d0, ps0, pd0, o0)
        out_store(c, o0, semo0)

        @pl.when(c + 2 < NCHUNK)
        def _():
            idx_wait(c + 2, is0, id0, semi0)
            split_idx(is0, id0, gs0, gd0, ps0, pd0)
            issue(gs0, gd0, s0, d0, sem0)

        wait(gs1, gd1, s1, d1, sem1)

        @pl.when(c + 3 < NCHUNK)
        def _():
            idx_stage(c + 3, is1, id1, semi1)

        @pl.when(cc >= 1)
        def _():
            out_wait(c - 1, o1, semo1)

        compute(s1, d1, ps1, pd1, o1)
        out_store(c + 1, o1, semo1)
        return carry

    lax.fori_loop(0, NCHUNK // 2, pair, 0)
    out_wait(NCHUNK - 2, o0, semo0)
    out_wait(NCHUNK - 1, o1, semo1)


def kernel(h, edges):
    hb = h.astype(jnp.bfloat16)
    packed = lax.bitcast_convert_type(hb.reshape(VP, NW_ROW, 2), jnp.int32)
    e32 = edges.astype(jnp.int32)
    pad = jnp.zeros((E_PAD - E,), jnp.int32)
    sidx = jnp.concatenate([e32[:, 0], pad])
    didx = jnp.concatenate([e32[:, 1], pad])
    return _dot_scores(packed, sidx, didx)[:E]


# speedup vs baseline: 3.2187x; 3.2187x over previous
import functools

import jax
import jax.numpy as jnp
from jax import lax
from jax.experimental import pallas as pl
from jax.experimental.pallas import tpu as pltpu
from jax.experimental.pallas import tpu_sc as plsc

D = 128
L = 16
NW = 32
V = 10000
E = 320000
E_PAD = 327680
EPW = E_PAD // NW
CHUNK = 80
NCHUNK = EPW // CHUNK
STRIPE = 624

_mesh = plsc.VectorSubcoreMesh(core_axis_name="c", subcore_axis_name="s")


@functools.partial(
    pl.kernel,
    out_type=jax.ShapeDtypeStruct((E_PAD,), jnp.float32),
    mesh=_mesh,
    scratch_types=[
        pltpu.VMEM_SHARED((V, D), jnp.float32),
        pltpu.VMEM((CHUNK,), jnp.int32),
        pltpu.VMEM((CHUNK,), jnp.int32),
        pltpu.VMEM((CHUNK,), jnp.int32),
        pltpu.VMEM((CHUNK,), jnp.int32),
        pltpu.VMEM((CHUNK, D), jnp.float32),
        pltpu.VMEM((CHUNK, D), jnp.float32),
        pltpu.VMEM((CHUNK, D), jnp.float32),
        pltpu.VMEM((CHUNK, D), jnp.float32),
        pltpu.VMEM((CHUNK,), jnp.float32),
        pltpu.VMEM((CHUNK,), jnp.float32),
        pltpu.SemaphoreType.DMA,
        pltpu.SemaphoreType.DMA,
        pltpu.SemaphoreType.DMA,
        pltpu.SemaphoreType.DMA,
        pltpu.SemaphoreType.DMA,
        pltpu.SemaphoreType.DMA,
    ],
    compiler_params=pltpu.CompilerParams(needs_layout_passes=False),
)
def _dot_scores(h_hbm, sidx_hbm, didx_hbm, out_hbm,
                tab_sp, is0, id0, is1, id1, s0, d0, s1, d1, o0, o1,
                sem0, sem1, semi0, semi1, semo0, semo1):
    sid = lax.axis_index("s")
    wid = sid * 2 + lax.axis_index("c")
    base = wid * EPW

    @pl.when(sid < 15)
    def _():
        pltpu.sync_copy(h_hbm.at[pl.ds(sid * STRIPE, STRIPE)],
                        tab_sp.at[pl.ds(sid * STRIPE, STRIPE)])

    @pl.when(sid == 15)
    def _():
        pltpu.sync_copy(h_hbm.at[pl.ds(15 * STRIPE, V - 15 * STRIPE)],
                        tab_sp.at[pl.ds(15 * STRIPE, V - 15 * STRIPE)])

    plsc.subcore_barrier()

    def idx_stage(c, isb, idb, sem, *, sync=False):
        sl = pl.ds(base + c * CHUNK, CHUNK)
        if sync:
            pltpu.sync_copy(sidx_hbm.at[sl], isb)
            pltpu.sync_copy(didx_hbm.at[sl], idb)
        else:
            pltpu.async_copy(sidx_hbm.at[sl], isb, sem)
            pltpu.async_copy(didx_hbm.at[sl], idb, sem)

    def idx_wait(c, isb, idb, sem):
        sl = pl.ds(base + c * CHUNK, CHUNK)
        pltpu.make_async_copy(sidx_hbm.at[sl], isb, sem).wait()
        pltpu.make_async_copy(didx_hbm.at[sl], idb, sem).wait()

    def issue(isb, idb, sbuf, dbuf, sem):
        pltpu.async_copy(tab_sp.at[isb], sbuf, sem)
        pltpu.async_copy(tab_sp.at[idb], dbuf, sem)

    def wait(isb, idb, sbuf, dbuf, sem):
        pltpu.make_async_copy(tab_sp.at[isb], sbuf, sem).wait()
        pltpu.make_async_copy(tab_sp.at[idb], dbuf, sem).wait()

    def out_store(c, ob, sem):
        pltpu.async_copy(ob, out_hbm.at[pl.ds(base + c * CHUNK, CHUNK)], sem)

    def out_wait(c, ob, sem):
        pltpu.make_async_copy(ob, out_hbm.at[pl.ds(base + c * CHUNK, CHUNK)],
                              sem).wait()

    def compute(sbuf, dbuf, ob):
        @plsc.parallel_loop(0, CHUNK, step=L)
        def edge_body(e0):
            lane = jnp.arange(L, dtype=jnp.int32)
            last = jnp.full((L,), L - 1, jnp.int32)
            res = jnp.zeros((L,), jnp.float32)
            for i in range(L):
                e = e0 + i
                acc = sbuf[e, pl.ds(0, L)] * dbuf[e, pl.ds(0, L)]
                for k in range(1, D // L):
                    acc = acc + sbuf[e, pl.ds(L * k, L)] * dbuf[e, pl.ds(L * k, L)]
                scn = plsc.cumsum(acc)
                res = jnp.where(lane == i, scn[last], res)
            ob[pl.ds(e0, L)] = res

    idx_stage(0, is0, id0, semi0, sync=True)
    issue(is0, id0, s0, d0, sem0)
    idx_stage(1, is1, id1, semi1)

    def pair(cc, carry):
        c = 2 * cc
        idx_wait(c + 1, is1, id1, semi1)
        issue(is1, id1, s1, d1, sem1)
        wait(is0, id0, s0, d0, sem0)

        @pl.when(c + 2 < NCHUNK)
        def _():
            idx_stage(c + 2, is0, id0, semi0)

        @pl.when(cc >= 1)
        def _():
            out_wait(c - 2, o0, semo0)

        compute(s0, d0, o0)
        out_store(c, o0, semo0)

        @pl.when(c + 2 < NCHUNK)
        def _():
            idx_wait(c + 2, is0, id0, semi0)
            issue(is0, id0, s0, d0, sem0)

        wait(is1, id1, s1, d1, sem1)

        @pl.when(c + 3 < NCHUNK)
        def _():
            idx_stage(c + 3, is1, id1, semi1)

        @pl.when(cc >= 1)
        def _():
            out_wait(c - 1, o1, semo1)

        compute(s1, d1, o1)
        out_store(c + 1, o1, semo1)
        return carry

    lax.fori_loop(0, NCHUNK // 2, pair, 0)
    out_wait(NCHUNK - 2, o0, semo0)
    out_wait(NCHUNK - 1, o1, semo1)


def kernel(h, edges):
    e32 = edges.astype(jnp.int32)
    pad = jnp.zeros((E_PAD - E,), jnp.int32)
    sidx = jnp.concatenate([e32[:, 0], pad])
    didx = jnp.concatenate([e32[:, 1], pad])
    return _dot_scores(h, sidx, didx)[:E]
